# baseline (speedup 1.0000x reference)
import functools

import jax
import jax.numpy as jnp
from jax import lax
from jax.experimental import pallas as pl
from jax.experimental.pallas import tpu as pltpu
from jax.experimental.pallas import tpu_sc as plsc

N = 10000
E = 320000
EPAD = 327680
D = 128

NC = 2
NS = 16
NW = NC * NS
EP = EPAD // NW
C = 128
CH = EP // C
EPS = EPAD // NS
CHS = EPS // C
DH = D // NC
NP = 10240
RP = NP // NS
ZR = 128
DW = 16

@functools.cache
def _build_deg_kernel():
    mesh = plsc.VectorSubcoreMesh(
        core_axis_name="c", subcore_axis_name="s",
        num_cores=NC, num_subcores=NS)
    return pl.kernel(
        _deg_body,
        out_type=jax.ShapeDtypeStruct((NP, D), jnp.float32),
        mesh=mesh,
        compiler_params=pltpu.CompilerParams(use_tc_tiling_on_sc=False),
        scratch_types=[
            pltpu.VMEM((CH, C), jnp.int32),
            pltpu.VMEM((C, DW), jnp.float32),
            pltpu.VMEM((ZR, DW), jnp.float32),
            pltpu.VMEM_SHARED((NP, DW), jnp.float32),
        ],
    )


def _deg_body(col_hbm, out_hbm, colbuf, ones_v, zrow, acc):
    c = lax.axis_index("c")
    s = lax.axis_index("s")
    wid = c * NS + s
    pltpu.sync_copy(col_hbm.at[wid], colbuf)

    def fill(k, _):
        zrow[k, :] = jnp.zeros((DW,), jnp.float32)
        return _

    lax.fori_loop(0, ZR, fill, 0)

    def fill1(k, _):
        ones_v[k, :] = jnp.ones((DW,), jnp.float32)
        return _

    lax.fori_loop(0, C, fill1, 0)

    base = s * RP
    for r in range(RP // ZR):
        pltpu.sync_copy(zrow, acc.at[pl.ds(base + r * ZR, ZR)])
    plsc.subcore_barrier()

    def step(j, _):
        pltpu.sync_copy(ones_v, acc.at[colbuf.at[j]], add=True)
        return _

    lax.fori_loop(0, CH, step, 0)
    plsc.subcore_barrier()
    pltpu.sync_copy(acc.at[pl.ds(base, RP)],
                    out_hbm.at[pl.ds(base, RP), pl.ds(c * DW, DW)])


@functools.cache
def _build_scatter_kernel():
    mesh = plsc.VectorSubcoreMesh(
        core_axis_name="c", subcore_axis_name="s",
        num_cores=NC, num_subcores=NS)
    return pl.kernel(
        _scatter_body,
        out_type=jax.ShapeDtypeStruct((NP, D), jnp.float32),
        mesh=mesh,
        compiler_params=pltpu.CompilerParams(use_tc_tiling_on_sc=False),
        scratch_types=[
            pltpu.VMEM((CHS, C), jnp.int32),
            pltpu.VMEM((CHS, C), jnp.int32),
            pltpu.VMEM((4, C, DH), jnp.float32),
            pltpu.VMEM((ZR, DH), jnp.float32),
            pltpu.VMEM_SHARED((NP, DH), jnp.float32),
            pltpu.SemaphoreType.DMA,
            pltpu.SemaphoreType.DMA,
            pltpu.SemaphoreType.DMA,
            pltpu.SemaphoreType.DMA,
        ],
    )


def _scatter_body(row_hbm, col_hbm, y_hbm, out_hbm,
                  rowbuf, colbuf, rows_v, zrow, acc, g0, g1, s0, s1):
    c = lax.axis_index("c")
    s = lax.axis_index("s")
    pltpu.sync_copy(row_hbm.at[s], rowbuf)
    pltpu.sync_copy(col_hbm.at[s], colbuf)

    SEG = C // 16
    def xform(k, _):
        i = k // SEG
        o = (k % SEG) * 16
        v = rowbuf[i, pl.ds(o, 16)]
        rowbuf[i, pl.ds(o, 16)] = v * 2 + c
        return _

    lax.fori_loop(0, CHS * SEG, xform, 0)

    ZSEG = DH // 16
    def fill_all(k, _):
        zrow[k // ZSEG, pl.ds((k % ZSEG) * 16, 16)] = jnp.zeros((16,), jnp.float32)
        return _

    lax.fori_loop(0, ZR * ZSEG, fill_all, 0)
    base = s * RP
    for r in range(RP // ZR):
        pltpu.sync_copy(zrow, acc.at[pl.ds(base + r * ZR, ZR)])
    plsc.subcore_barrier()

    gsems = (g0, g1)
    ssems = (s0, s1)

    pltpu.async_copy(y_hbm.at[rowbuf.at[0]], rows_v.at[0], g0)
    pltpu.async_copy(y_hbm.at[rowbuf.at[1]], rows_v.at[1], g1)

    def _drain(sem):
        pltpu.make_async_copy(y_hbm.at[rowbuf.at[0]], rows_v.at[0], sem).wait()

    def loop_body(j, _):
        for p in range(2):
            @pl.when(j % 2 == p)
            def _w(p=p):
                _drain(gsems[p])

                @pl.when(j >= 2)
                def _ws():
                    _drain(ssems[p])

        for k in range(4):
            @pl.when(j % 4 == k)
            def _go(k=k):
                pltpu.async_copy(rows_v.at[k], acc.at[colbuf.at[j]],
                                 ssems[k % 2], add=True)

                @pl.when(j + 2 < CHS)
                def _pf():
                    pltpu.async_copy(y_hbm.at[rowbuf.at[j + 2]],
                                     rows_v.at[(k + 2) % 4], gsems[k % 2])

        return _

    lax.fori_loop(0, CHS, loop_body, 0)
    _drain(s0)
    _drain(s1)
    plsc.subcore_barrier()
    pltpu.sync_copy(acc.at[pl.ds(base, RP)],
                    out_hbm.at[pl.ds(base, RP), pl.ds(c * DH, DH)])


_R = 1000


def _tc1_body(x_ref, w_ref, b_ref, degp_ref, y1_ref):
    deg = degp_ref[:, 0:1] + degp_ref[:, DW:DW + 1] + 1.0
    dinv = lax.rsqrt(deg)
    h = jnp.dot(x_ref[...], w_ref[...], preferred_element_type=jnp.float32)
    h = jnp.maximum(h + b_ref[...], 0.0)
    y1_ref[...] = dinv * h


def _tc2_body(sp_ref, y_ref, degp_ref, w_ref, b_ref, out_ref, y2_ref):
    deg = degp_ref[:, 0:1] + degp_ref[:, DW:DW + 1] + 1.0
    dinv = lax.rsqrt(deg)
    agg = dinv * (sp_ref[...] + y_ref[...])
    h = jnp.dot(agg, w_ref[...], preferred_element_type=jnp.float32)
    h = jnp.maximum(h + b_ref[...], 0.0)
    out_ref[...] = h
    y2_ref[...] = dinv * h


def _tc1(x, w, b2, degp):
    grid = N // _R
    return pl.pallas_call(
        _tc1_body,
        grid=(grid,),
        in_specs=[
            pl.BlockSpec((_R, D), lambda i: (i, 0)),
            pl.BlockSpec((D, D), lambda i: (0, 0)),
            pl.BlockSpec((1, D), lambda i: (0, 0)),
            pl.BlockSpec((_R, D), lambda i: (i, 0)),
        ],
        out_specs=pl.BlockSpec((_R, D), lambda i: (i, 0)),
        out_shape=jax.ShapeDtypeStruct((N, D), jnp.float32),
    )(x, w, b2, degp)


def _tc2(sp, y, degp, w, b2):
    grid = N // _R
    return pl.pallas_call(
        _tc2_body,
        grid=(grid,),
        in_specs=[
            pl.BlockSpec((_R, D), lambda i: (i, 0)),
            pl.BlockSpec((_R, D), lambda i: (i, 0)),
            pl.BlockSpec((_R, D), lambda i: (i, 0)),
            pl.BlockSpec((D, D), lambda i: (0, 0)),
            pl.BlockSpec((1, D), lambda i: (0, 0)),
        ],
        out_specs=[
            pl.BlockSpec((_R, D), lambda i: (i, 0)),
            pl.BlockSpec((_R, D), lambda i: (i, 0)),
        ],
        out_shape=[
            jax.ShapeDtypeStruct((N, D), jnp.float32),
            jax.ShapeDtypeStruct((N, D), jnp.float32),
        ],
    )(sp, y, degp, w, b2)


def kernel(input, edge_index, batch, W, b, W0, b0, W1, b1):
    npad = EPAD - E
    rowp = jnp.concatenate([edge_index[0], jnp.zeros((npad,), jnp.int32)])
    pad_cols = N + (jnp.arange(npad, dtype=jnp.int32) % (NP - N))
    colp = jnp.concatenate([edge_index[1], pad_cols])
    col_deg = colp.reshape(NW, CH, C)
    row = rowp.reshape(NS, CHS, C)
    col = colp.reshape(NS, CHS, C)
    b2 = b.reshape(1, D)
    b02 = b0.reshape(1, D)
    b12 = b1.reshape(1, D)

    deg_k = _build_deg_kernel()
    scat_k = _build_scatter_kernel()
    degp = deg_k(col_deg)
    y1 = _tc1(input, W, b2, degp)
    s1 = scat_k(row, col, y1.reshape(NC * N, DH))
    h2, y2 = _tc2(s1, y1, degp, W0, b02)
    s2 = scat_k(row, col, y2.reshape(NC * N, DH))
    h3, _ = _tc2(s2, y2, degp, W1, b12)
    return h3


# speedup vs baseline: 2.4884x; 2.4884x over previous
import functools

import jax
import jax.numpy as jnp
from jax import lax
from jax.experimental import pallas as pl
from jax.experimental.pallas import tpu as pltpu
from jax.experimental.pallas import tpu_sc as plsc

N = 10000
E = 320000
D = 128

NC = 2
NS = 16
NW = NC * NS
EP = E // NW
C = 80
CH = EP // C
EPS = E // NS
CHS = EPS // C
DH = D // NC
NP = 10240
RP = NP // NS
ZR = 128
DW = 16

@functools.cache
def _build_deg_kernel():
    mesh = plsc.VectorSubcoreMesh(
        core_axis_name="c", subcore_axis_name="s",
        num_cores=NC, num_subcores=NS)
    return pl.kernel(
        _deg_body,
        out_type=jax.ShapeDtypeStruct((NP, D), jnp.float32),
        mesh=mesh,
        compiler_params=pltpu.CompilerParams(use_tc_tiling_on_sc=False),
        scratch_types=[
            pltpu.VMEM((CH, C), jnp.int32),
            pltpu.VMEM((C, DW), jnp.float32),
            pltpu.VMEM((ZR, DW), jnp.float32),
            pltpu.VMEM_SHARED((NP, DW), jnp.float32),
        ],
    )


def _deg_body(col_hbm, out_hbm, colbuf, ones_v, zrow, acc):
    c = lax.axis_index("c")
    s = lax.axis_index("s")
    wid = c * NS + s
    pltpu.sync_copy(col_hbm.at[wid], colbuf)

    def fill(k, _):
        zrow[k, :] = jnp.zeros((DW,), jnp.float32)
        return _

    lax.fori_loop(0, ZR, fill, 0)

    def fill1(k, _):
        ones_v[k, :] = jnp.ones((DW,), jnp.float32)
        return _

    lax.fori_loop(0, C, fill1, 0)

    base = s * RP
    for r in range(RP // ZR):
        pltpu.sync_copy(zrow, acc.at[pl.ds(base + r * ZR, ZR)])
    plsc.subcore_barrier()

    def step(j, _):
        pltpu.sync_copy(ones_v, acc.at[colbuf.at[j]], add=True)
        return _

    lax.fori_loop(0, CH, step, 0)
    plsc.subcore_barrier()
    pltpu.sync_copy(acc.at[pl.ds(base, RP)],
                    out_hbm.at[pl.ds(base, RP), pl.ds(c * DW, DW)])


@functools.cache
def _build_scatter_kernel():
    mesh = plsc.VectorSubcoreMesh(
        core_axis_name="c", subcore_axis_name="s",
        num_cores=NC, num_subcores=NS)
    return pl.kernel(
        _scatter_body,
        out_type=jax.ShapeDtypeStruct((NP, D), jnp.float32),
        mesh=mesh,
        compiler_params=pltpu.CompilerParams(use_tc_tiling_on_sc=False),
        scratch_types=[
            pltpu.VMEM((CHS, C), jnp.int32),
            pltpu.VMEM((CHS, C), jnp.int32),
            pltpu.VMEM((4, C, DH), jnp.float32),
            pltpu.VMEM((ZR, DH), jnp.float32),
            pltpu.VMEM_SHARED((NP, DH), jnp.float32),
            pltpu.SemaphoreType.DMA,
            pltpu.SemaphoreType.DMA,
            pltpu.SemaphoreType.DMA,
            pltpu.SemaphoreType.DMA,
        ],
    )


def _scatter_body(row_hbm, col_hbm, y_hbm, out_hbm,
                  rowbuf, colbuf, rows_v, zrow, acc, g0, g1, s0, s1):
    c = lax.axis_index("c")
    s = lax.axis_index("s")
    gsems = (g0, g1)
    ssems = (s0, s1)

    pltpu.sync_copy(row_hbm.at[c, s], rowbuf)
    pltpu.async_copy(y_hbm.at[rowbuf.at[0]], rows_v.at[0], g0)
    pltpu.async_copy(y_hbm.at[rowbuf.at[1]], rows_v.at[1], g1)
    pltpu.sync_copy(col_hbm.at[s], colbuf)

    ZSEG = DH // 16
    def fill_all(k, _):
        zrow[k // ZSEG, pl.ds((k % ZSEG) * 16, 16)] = jnp.zeros((16,), jnp.float32)
        return _

    lax.fori_loop(0, ZR * ZSEG, fill_all, 0)
    base = s * RP
    for r in range(RP // ZR):
        pltpu.sync_copy(zrow, acc.at[pl.ds(base + r * ZR, ZR)])
    plsc.subcore_barrier()

    def _drain(sem):
        pltpu.make_async_copy(y_hbm.at[rowbuf.at[0]], rows_v.at[0], sem).wait()

    def loop_body(j, _):
        for p in range(2):
            @pl.when(j % 2 == p)
            def _w(p=p):
                _drain(gsems[p])

                @pl.when(j >= 2)
                def _ws():
                    _drain(ssems[p])

        for k in range(4):
            @pl.when(j % 4 == k)
            def _go(k=k):
                pltpu.async_copy(rows_v.at[k], acc.at[colbuf.at[j]],
                                 ssems[k % 2], add=True)

                @pl.when(j + 2 < CHS)
                def _pf():
                    pltpu.async_copy(y_hbm.at[rowbuf.at[j + 2]],
                                     rows_v.at[(k + 2) % 4], gsems[k % 2])

        return _

    lax.fori_loop(0, CHS, loop_body, 0)
    _drain(s0)
    _drain(s1)
    plsc.subcore_barrier()
    pltpu.sync_copy(acc.at[pl.ds(base, RP)],
                    out_hbm.at[pl.ds(base, RP), pl.ds(c * DH, DH)])


_R = 1000


def _tc1_body(x_ref, w_ref, b_ref, degp_ref, y1_ref):
    deg = degp_ref[:, 0:1] + degp_ref[:, DW:DW + 1] + 1.0
    dinv = lax.rsqrt(deg)
    h = jnp.dot(x_ref[...], w_ref[...], preferred_element_type=jnp.float32)
    h = jnp.maximum(h + b_ref[...], 0.0)
    y1_ref[...] = dinv * h


def _tc2_body(sp_ref, y_ref, degp_ref, w_ref, b_ref, out_ref, y2_ref):
    deg = degp_ref[:, 0:1] + degp_ref[:, DW:DW + 1] + 1.0
    dinv = lax.rsqrt(deg)
    agg = dinv * (sp_ref[...] + y_ref[...])
    h = jnp.dot(agg, w_ref[...], preferred_element_type=jnp.float32)
    h = jnp.maximum(h + b_ref[...], 0.0)
    out_ref[...] = h
    y2_ref[...] = dinv * h


def _tc1(x, w, b2, degp):
    grid = N // _R
    return pl.pallas_call(
        _tc1_body,
        grid=(grid,),
        in_specs=[
            pl.BlockSpec((_R, D), lambda i: (i, 0)),
            pl.BlockSpec((D, D), lambda i: (0, 0)),
            pl.BlockSpec((1, D), lambda i: (0, 0)),
            pl.BlockSpec((_R, D), lambda i: (i, 0)),
        ],
        out_specs=pl.BlockSpec((_R, D), lambda i: (i, 0)),
        out_shape=jax.ShapeDtypeStruct((N, D), jnp.float32),
    )(x, w, b2, degp)


def _tc2(sp, y, degp, w, b2):
    grid = N // _R
    return pl.pallas_call(
        _tc2_body,
        grid=(grid,),
        in_specs=[
            pl.BlockSpec((_R, D), lambda i: (i, 0)),
            pl.BlockSpec((_R, D), lambda i: (i, 0)),
            pl.BlockSpec((_R, D), lambda i: (i, 0)),
            pl.BlockSpec((D, D), lambda i: (0, 0)),
            pl.BlockSpec((1, D), lambda i: (0, 0)),
        ],
        out_specs=[
            pl.BlockSpec((_R, D), lambda i: (i, 0)),
            pl.BlockSpec((_R, D), lambda i: (i, 0)),
        ],
        out_shape=[
            jax.ShapeDtypeStruct((N, D), jnp.float32),
            jax.ShapeDtypeStruct((N, D), jnp.float32),
        ],
    )(sp, y, degp, w, b2)


def kernel(input, edge_index, batch, W, b, W0, b0, W1, b1):
    col_deg = edge_index[1].reshape(NW, CH, C)
    r2 = edge_index[0] * 2
    row = jnp.stack([r2, r2 + 1]).reshape(NC, NS, CHS, C)
    col = edge_index[1].reshape(NS, CHS, C)
    b2 = b.reshape(1, D)
    b02 = b0.reshape(1, D)
    b12 = b1.reshape(1, D)

    deg_k = _build_deg_kernel()
    scat_k = _build_scatter_kernel()
    degp = deg_k(col_deg)
    y1 = _tc1(input, W, b2, degp)
    s1 = scat_k(row, col, y1.reshape(NC * N, DH))
    h2, y2 = _tc2(s1, y1, degp, W0, b02)
    s2 = scat_k(row, col, y2.reshape(NC * N, DH))
    h3, _ = _tc2(s2, y2, degp, W1, b12)
    return h3


# speedup vs baseline: 2.4964x; 1.0032x over previous
import functools

import jax
import jax.numpy as jnp
from jax import lax
from jax.experimental import pallas as pl
from jax.experimental.pallas import tpu as pltpu
from jax.experimental.pallas import tpu_sc as plsc

N = 10000
E = 320000
D = 128

NC = 2
NS = 16
NW = NC * NS
EP = E // NW
C = 80
CH = EP // C
EPS = E // NS
CHS = EPS // C
DH = D // NC
NP = 10240
RP = NP // NS
ZR = 128
DW = 16

@functools.cache
def _build_deg_kernel():
    mesh = plsc.VectorSubcoreMesh(
        core_axis_name="c", subcore_axis_name="s",
        num_cores=NC, num_subcores=NS)
    return pl.kernel(
        _deg_body,
        out_type=jax.ShapeDtypeStruct((NP, D), jnp.float32),
        mesh=mesh,
        compiler_params=pltpu.CompilerParams(use_tc_tiling_on_sc=False),
        scratch_types=[
            pltpu.VMEM((CH, C), jnp.int32),
            pltpu.VMEM((C, DW), jnp.float32),
            pltpu.VMEM((ZR, DW), jnp.float32),
            pltpu.VMEM_SHARED((NP, DW), jnp.float32),
        ],
    )


def _deg_body(col_hbm, out_hbm, colbuf, ones_v, zrow, acc):
    c = lax.axis_index("c")
    s = lax.axis_index("s")
    wid = c * NS + s
    pltpu.sync_copy(col_hbm.at[wid], colbuf)

    def fill(k, _):
        zrow[k, :] = jnp.zeros((DW,), jnp.float32)
        return _

    lax.fori_loop(0, ZR, fill, 0)

    def fill1(k, _):
        ones_v[k, :] = jnp.ones((DW,), jnp.float32)
        return _

    lax.fori_loop(0, C, fill1, 0)

    base = s * RP
    for r in range(RP // ZR):
        pltpu.sync_copy(zrow, acc.at[pl.ds(base + r * ZR, ZR)])
    plsc.subcore_barrier()

    def step(j, _):
        pltpu.sync_copy(ones_v, acc.at[colbuf.at[j]], add=True)
        return _

    lax.fori_loop(0, CH, step, 0)
    plsc.subcore_barrier()
    pltpu.sync_copy(acc.at[pl.ds(base, RP)],
                    out_hbm.at[pl.ds(base, RP), pl.ds(c * DW, DW)])


@functools.cache
def _build_scatter_kernel():
    mesh = plsc.VectorSubcoreMesh(
        core_axis_name="c", subcore_axis_name="s",
        num_cores=NC, num_subcores=NS)
    return pl.kernel(
        _scatter_body,
        out_type=jax.ShapeDtypeStruct((NP, D), jnp.float32),
        mesh=mesh,
        compiler_params=pltpu.CompilerParams(use_tc_tiling_on_sc=False),
        scratch_types=[
            pltpu.VMEM((CHS, C), jnp.int32),
            pltpu.VMEM((CHS, C), jnp.int32),
            pltpu.VMEM((4, C, DH), jnp.float32),
            pltpu.VMEM((ZR, DH), jnp.float32),
            pltpu.VMEM_SHARED((NP, DH), jnp.float32),
            pltpu.SemaphoreType.DMA,
            pltpu.SemaphoreType.DMA,
            pltpu.SemaphoreType.DMA,
            pltpu.SemaphoreType.DMA,
        ],
    )


def _scatter_body(row_hbm, col_hbm, y_hbm, out_hbm,
                  rowbuf, colbuf, rows_v, zrow, acc, g0, g1, s0, s1):
    c = lax.axis_index("c")
    s = lax.axis_index("s")
    gsems = (g0, g1)
    ssems = (s0, s1)

    pltpu.sync_copy(row_hbm.at[c, s], rowbuf)
    pltpu.async_copy(y_hbm.at[rowbuf.at[0]], rows_v.at[0], g0)
    pltpu.async_copy(y_hbm.at[rowbuf.at[1]], rows_v.at[1], g1)
    pltpu.sync_copy(col_hbm.at[s], colbuf)

    ZSEG = DH // 16
    def fill_all(k, _):
        zrow[k // ZSEG, pl.ds((k % ZSEG) * 16, 16)] = jnp.zeros((16,), jnp.float32)
        return _

    lax.fori_loop(0, ZR * ZSEG, fill_all, 0)
    base = s * RP
    for r in range(RP // ZR):
        pltpu.sync_copy(zrow, acc.at[pl.ds(base + r * ZR, ZR)])
    plsc.subcore_barrier()

    def _drain(sem):
        pltpu.make_async_copy(y_hbm.at[rowbuf.at[0]], rows_v.at[0], sem).wait()

    def loop_body(j, _):
        for p in range(2):
            @pl.when(j % 2 == p)
            def _w(p=p):
                _drain(gsems[p])

                @pl.when(j >= 2)
                def _ws():
                    _drain(ssems[p])

        for k in range(4):
            @pl.when(j % 4 == k)
            def _go(k=k):
                pltpu.async_copy(rows_v.at[k], acc.at[colbuf.at[j]],
                                 ssems[k % 2], add=True)

                @pl.when(j + 2 < CHS)
                def _pf():
                    pltpu.async_copy(y_hbm.at[rowbuf.at[j + 2]],
                                     rows_v.at[(k + 2) % 4], gsems[k % 2])

        return _

    lax.fori_loop(0, CHS, loop_body, 0)
    _drain(s0)
    _drain(s1)
    plsc.subcore_barrier()
    pltpu.sync_copy(acc.at[pl.ds(base, RP)],
                    out_hbm.at[pl.ds(base, RP), pl.ds(c * DH, DH)])


_R = 1000


def _tc1_body(x_ref, w_ref, b_ref, degp_ref, y1_ref):
    deg = degp_ref[:, 0:1] + degp_ref[:, DW:DW + 1] + 1.0
    dinv = lax.rsqrt(deg)
    h = jnp.dot(x_ref[...], w_ref[...], preferred_element_type=jnp.float32)
    h = jnp.maximum(h + b_ref[...], 0.0)
    y1_ref[...] = dinv * h


def _tc2_body(sp_ref, y_ref, degp_ref, w_ref, b_ref, out_ref, y2_ref):
    deg = degp_ref[:, 0:1] + degp_ref[:, DW:DW + 1] + 1.0
    dinv = lax.rsqrt(deg)
    agg = dinv * (sp_ref[...] + y_ref[...])
    h = jnp.dot(agg, w_ref[...], preferred_element_type=jnp.float32)
    h = jnp.maximum(h + b_ref[...], 0.0)
    out_ref[...] = h
    y2_ref[...] = dinv * h


def _tc1(x, w, b2, degp):
    grid = N // _R
    return pl.pallas_call(
        _tc1_body,
        grid=(grid,),
        in_specs=[
            pl.BlockSpec((_R, D), lambda i: (i, 0)),
            pl.BlockSpec((D, D), lambda i: (0, 0)),
            pl.BlockSpec((1, D), lambda i: (0, 0)),
            pl.BlockSpec((_R, D), lambda i: (i, 0)),
        ],
        out_specs=pl.BlockSpec((_R, D), lambda i: (i, 0)),
        out_shape=jax.ShapeDtypeStruct((N, D), jnp.float32),
    )(x, w, b2, degp)


def _tc3_body(sp_ref, y_ref, degp_ref, w_ref, b_ref, out_ref):
    deg = degp_ref[:, 0:1] + degp_ref[:, DW:DW + 1] + 1.0
    dinv = lax.rsqrt(deg)
    agg = dinv * (sp_ref[...] + y_ref[...])
    h = jnp.dot(agg, w_ref[...], preferred_element_type=jnp.float32)
    out_ref[...] = jnp.maximum(h + b_ref[...], 0.0)


def _tc2(sp, y, degp, w, b2):
    grid = N // _R
    return pl.pallas_call(
        _tc2_body,
        grid=(grid,),
        in_specs=[
            pl.BlockSpec((_R, D), lambda i: (i, 0)),
            pl.BlockSpec((_R, D), lambda i: (i, 0)),
            pl.BlockSpec((_R, D), lambda i: (i, 0)),
            pl.BlockSpec((D, D), lambda i: (0, 0)),
            pl.BlockSpec((1, D), lambda i: (0, 0)),
        ],
        out_specs=[
            pl.BlockSpec((_R, D), lambda i: (i, 0)),
            pl.BlockSpec((_R, D), lambda i: (i, 0)),
        ],
        out_shape=[
            jax.ShapeDtypeStruct((N, D), jnp.float32),
            jax.ShapeDtypeStruct((N, D), jnp.float32),
        ],
    )(sp, y, degp, w, b2)


def _tc3(sp, y, degp, w, b2):
    grid = N // _R
    return pl.pallas_call(
        _tc3_body,
        grid=(grid,),
        in_specs=[
            pl.BlockSpec((_R, D), lambda i: (i, 0)),
            pl.BlockSpec((_R, D), lambda i: (i, 0)),
            pl.BlockSpec((_R, D), lambda i: (i, 0)),
            pl.BlockSpec((D, D), lambda i: (0, 0)),
            pl.BlockSpec((1, D), lambda i: (0, 0)),
        ],
        out_specs=pl.BlockSpec((_R, D), lambda i: (i, 0)),
        out_shape=jax.ShapeDtypeStruct((N, D), jnp.float32),
    )(sp, y, degp, w, b2)


def kernel(input, edge_index, batch, W, b, W0, b0, W1, b1):
    col_deg = edge_index[1].reshape(NW, CH, C)
    r2 = edge_index[0] * 2
    row = jnp.stack([r2, r2 + 1]).reshape(NC, NS, CHS, C)
    col = edge_index[1].reshape(NS, CHS, C)
    b2 = b.reshape(1, D)
    b02 = b0.reshape(1, D)
    b12 = b1.reshape(1, D)

    deg_k = _build_deg_kernel()
    scat_k = _build_scatter_kernel()
    degp = deg_k(col_deg)
    y1 = _tc1(input, W, b2, degp)
    s1 = scat_k(row, col, y1.reshape(NC * N, DH))
    h2, y2 = _tc2(s1, y1, degp, W0, b02)
    s2 = scat_k(row, col, y2.reshape(NC * N, DH))
    return _tc3(s2, y2, degp, W1, b12)
